# baseline (speedup 1.0000x reference)
import functools

import jax
import jax.numpy as jnp
from jax import lax
from jax.experimental import pallas as pl
from jax.experimental.pallas import tpu as pltpu
from jax.experimental.pallas import tpu_sc as plsc

_B, _S, _D = 4, 8192, 1024
_NC, _NS = 2, 16
_NW = _NC * _NS
_CH = 16
_LANES = 16
_SPW = _S // _NW
_NCHUNK = _SPW // _CH
_NT = _NCHUNK * _B
_NIN = 4
_NOUT = 2


def _sc_body(x_hbm, pos_hbm, out_hbm, *scratch):
    pos_bufs = list(scratch[0:2])
    x_bufs = list(scratch[2:2 + _NIN])
    o_bufs = list(scratch[2 + _NIN:2 + _NIN + _NOUT])
    nb = 2 + _NIN + _NOUT
    pos_sems = list(scratch[nb + 0:nb + 2])
    in_sems = list(scratch[nb + 2:nb + 2 + _NIN])
    out_sems = list(scratch[nb + 2 + _NIN:nb + 2 + _NIN + _NOUT])

    wid = lax.axis_index("s") * _NC + lax.axis_index("c")
    base = wid * _SPW

    def pos_src(ci):
        return pos_hbm.at[pl.ds(base + ci * _CH, _CH)]

    def x_src(ci, b):
        return x_hbm.at[b, pl.ds(base + ci * _CH, _CH)]

    def out_dst(ci, b):
        return out_hbm.at[b, pl.ds(base + ci * _CH, _CH)]

    pltpu.async_copy(pos_src(0), pos_bufs[0], pos_sems[0])
    pltpu.async_copy(pos_src(1), pos_bufs[1], pos_sems[1])
    for t in range(_NIN - 1):
        pltpu.async_copy(x_src(t // _B, t % _B), x_bufs[t % _NIN],
                         in_sems[t % _NIN])

    @pl.loop(0, _NCHUNK, step=2)
    def _pair(ci0):
        for k in range(2 * _B):
            ci = ci0 + k // _B
            b = k % _B
            islot = k % _NIN
            oslot = k % _NOUT
            pb = k // _B
            t = ci0 * _B + k

            if k % _B == 0:
                pltpu.make_async_copy(pos_src(ci), pos_bufs[pb],
                                      pos_sems[pb]).wait()
            pltpu.make_async_copy(x_src(ci, b), x_bufs[islot],
                                  in_sems[islot]).wait()

            @pl.when(t >= _NOUT)
            def _():
                pltpu.make_async_copy(o_bufs[oslot], out_dst(ci, b),
                                      out_sems[oslot]).wait()

            xb, ob, pbuf = x_bufs[islot], o_bufs[oslot], pos_bufs[pb]

            @plsc.parallel_loop(0, _CH * (_D // _LANES), unroll=8)
            def _elem(i):
                r = i // (_D // _LANES)
                sl = pl.ds((i % (_D // _LANES)) * _LANES, _LANES)
                ob[r, sl] = xb[r, sl] + pbuf[r, sl]

            pltpu.async_copy(ob, out_dst(ci, b), out_sems[oslot])

            s2 = (k + _NIN - 1) % _NIN
            ci2 = ci0 + (k + _NIN - 1) // _B
            b2 = (k + _NIN - 1) % _B

            @pl.when(t + _NIN - 1 < _NT)
            def _():
                pltpu.async_copy(x_src(ci2, b2), x_bufs[s2], in_sems[s2])

            if k % _B == _B - 1:
                @pl.when(ci + 2 < _NCHUNK)
                def _():
                    pltpu.async_copy(pos_src(ci + 2), pos_bufs[pb],
                                     pos_sems[pb])

    last = _NCHUNK - 1
    for j in range(_NOUT):
        pltpu.make_async_copy(o_bufs[j], out_dst(last, j),
                              out_sems[j]).wait()


_sc_call = functools.partial(
    pl.kernel,
    out_type=jax.ShapeDtypeStruct((_B, _S, _D), jnp.float32),
    mesh=plsc.VectorSubcoreMesh(
        core_axis_name="c", subcore_axis_name="s",
        num_cores=_NC, num_subcores=_NS,
    ),
    scratch_types=(
        [pltpu.VMEM((_CH, _D), jnp.float32)] * (2 + _NIN + _NOUT)
        + [pltpu.SemaphoreType.DMA] * (2 + _NIN + _NOUT)
    ),
    compiler_params=pltpu.CompilerParams(use_tc_tiling_on_sc=True),
)(_sc_body)


def kernel(x, pos_table):
    B, S, D = x.shape
    return _sc_call(x, pos_table[:S])


# speedup vs baseline: 1.0166x; 1.0166x over previous
import functools

import jax
import jax.numpy as jnp
from jax import lax
from jax.experimental import pallas as pl
from jax.experimental.pallas import tpu as pltpu
from jax.experimental.pallas import tpu_sc as plsc

_B, _S, _D = 4, 8192, 1024
_NC, _NS = 2, 16
_NW = _NC * _NS
_CH = 8
_LANES = 16
_SPW = _S // _NW
_NCHUNK = _SPW // _CH
_NT = _NCHUNK * _B
_NIN = 8
_NOUT = 4


def _sc_body(x_hbm, pos_hbm, out_hbm, *scratch):
    pos_bufs = list(scratch[0:2])
    x_bufs = list(scratch[2:2 + _NIN])
    o_bufs = list(scratch[2 + _NIN:2 + _NIN + _NOUT])
    nb = 2 + _NIN + _NOUT
    pos_sems = list(scratch[nb + 0:nb + 2])
    in_sems = list(scratch[nb + 2:nb + 2 + _NIN])
    out_sems = list(scratch[nb + 2 + _NIN:nb + 2 + _NIN + _NOUT])

    wid = lax.axis_index("s") * _NC + lax.axis_index("c")
    base = wid * _SPW

    def pos_src(ci):
        return pos_hbm.at[pl.ds(base + ci * _CH, _CH)]

    def x_src(ci, b):
        return x_hbm.at[b, pl.ds(base + ci * _CH, _CH)]

    def out_dst(ci, b):
        return out_hbm.at[b, pl.ds(base + ci * _CH, _CH)]

    pltpu.async_copy(pos_src(0), pos_bufs[0], pos_sems[0])
    pltpu.async_copy(pos_src(1), pos_bufs[1], pos_sems[1])
    for t in range(_NIN - 1):
        pltpu.async_copy(x_src(t // _B, t % _B), x_bufs[t % _NIN],
                         in_sems[t % _NIN])

    @pl.loop(0, _NCHUNK, step=2)
    def _pair(ci0):
        for k in range(2 * _B):
            ci = ci0 + k // _B
            b = k % _B
            islot = k % _NIN
            oslot = k % _NOUT
            pb = k // _B
            t = ci0 * _B + k

            if k % _B == 0:
                pltpu.make_async_copy(pos_src(ci), pos_bufs[pb],
                                      pos_sems[pb]).wait()
            pltpu.make_async_copy(x_src(ci, b), x_bufs[islot],
                                  in_sems[islot]).wait()

            @pl.when(t >= _NOUT)
            def _():
                pltpu.make_async_copy(o_bufs[oslot], out_dst(ci, b),
                                      out_sems[oslot]).wait()

            xb, ob, pbuf = x_bufs[islot], o_bufs[oslot], pos_bufs[pb]

            @plsc.parallel_loop(0, _CH * (_D // _LANES), unroll=8)
            def _elem(i):
                r = i // (_D // _LANES)
                sl = pl.ds((i % (_D // _LANES)) * _LANES, _LANES)
                ob[r, sl] = xb[r, sl] + pbuf[r, sl]

            pltpu.async_copy(ob, out_dst(ci, b), out_sems[oslot])

            s2 = (k + _NIN - 1) % _NIN
            ci2 = ci0 + (k + _NIN - 1) // _B
            b2 = (k + _NIN - 1) % _B

            @pl.when(t + _NIN - 1 < _NT)
            def _():
                pltpu.async_copy(x_src(ci2, b2), x_bufs[s2], in_sems[s2])

            if k % _B == _B - 1:
                @pl.when(ci + 2 < _NCHUNK)
                def _():
                    pltpu.async_copy(pos_src(ci + 2), pos_bufs[pb],
                                     pos_sems[pb])

    last = _NCHUNK - 1
    for j in range(_NOUT):
        pltpu.make_async_copy(o_bufs[j], out_dst(last, j),
                              out_sems[j]).wait()


_sc_call = functools.partial(
    pl.kernel,
    out_type=jax.ShapeDtypeStruct((_B, _S, _D), jnp.float32),
    mesh=plsc.VectorSubcoreMesh(
        core_axis_name="c", subcore_axis_name="s",
        num_cores=_NC, num_subcores=_NS,
    ),
    scratch_types=(
        [pltpu.VMEM((_CH, _D), jnp.float32)] * (2 + _NIN + _NOUT)
        + [pltpu.SemaphoreType.DMA] * (2 + _NIN + _NOUT)
    ),
    compiler_params=pltpu.CompilerParams(use_tc_tiling_on_sc=True),
)(_sc_body)


def kernel(x, pos_table):
    B, S, D = x.shape
    return _sc_call(x, pos_table[:S])


# speedup vs baseline: 1.0188x; 1.0022x over previous
import functools

import jax
import jax.numpy as jnp
from jax import lax
from jax.experimental import pallas as pl
from jax.experimental.pallas import tpu as pltpu
from jax.experimental.pallas import tpu_sc as plsc

_B, _S, _D = 4, 8192, 1024
_NC, _NS = 2, 16
_NW = _NC * _NS
_CH = 8
_LANES = 16
_SPW = _S // _NW
_NCHUNK = _SPW // _CH
_NT = _NCHUNK * _B
_NIN = 8
_NOUT = 4


def _sc_body(x_hbm, pos_hbm, out_hbm, *scratch):
    pos_bufs = list(scratch[0:2])
    x_bufs = list(scratch[2:2 + _NIN])
    o_bufs = list(scratch[2 + _NIN:2 + _NIN + _NOUT])
    nb = 2 + _NIN + _NOUT
    pos_sems = list(scratch[nb + 0:nb + 2])
    in_sems = list(scratch[nb + 2:nb + 2 + _NIN])
    out_sems = list(scratch[nb + 2 + _NIN:nb + 2 + _NIN + _NOUT])

    wid = lax.axis_index("c") * _NS + lax.axis_index("s")
    base = wid * _SPW

    def pos_src(ci):
        return pos_hbm.at[pl.ds(base + ci * _CH, _CH)]

    def x_src(ci, b):
        return x_hbm.at[b, pl.ds(base + ci * _CH, _CH)]

    def out_dst(ci, b):
        return out_hbm.at[b, pl.ds(base + ci * _CH, _CH)]

    pltpu.async_copy(pos_src(0), pos_bufs[0], pos_sems[0])
    pltpu.async_copy(pos_src(1), pos_bufs[1], pos_sems[1])
    for t in range(_NIN - 1):
        pltpu.async_copy(x_src(t // _B, t % _B), x_bufs[t % _NIN],
                         in_sems[t % _NIN])

    @pl.loop(0, _NCHUNK, step=2)
    def _pair(ci0):
        for k in range(2 * _B):
            ci = ci0 + k // _B
            b = k % _B
            islot = k % _NIN
            oslot = k % _NOUT
            pb = k // _B
            t = ci0 * _B + k

            if k % _B == 0:
                pltpu.make_async_copy(pos_src(ci), pos_bufs[pb],
                                      pos_sems[pb]).wait()
            pltpu.make_async_copy(x_src(ci, b), x_bufs[islot],
                                  in_sems[islot]).wait()

            @pl.when(t >= _NOUT)
            def _():
                pltpu.make_async_copy(o_bufs[oslot], out_dst(ci, b),
                                      out_sems[oslot]).wait()

            xb, ob, pbuf = x_bufs[islot], o_bufs[oslot], pos_bufs[pb]

            @plsc.parallel_loop(0, _CH * (_D // _LANES), unroll=8)
            def _elem(i):
                r = i // (_D // _LANES)
                sl = pl.ds((i % (_D // _LANES)) * _LANES, _LANES)
                ob[r, sl] = xb[r, sl] + pbuf[r, sl]

            pltpu.async_copy(ob, out_dst(ci, b), out_sems[oslot])

            s2 = (k + _NIN - 1) % _NIN
            ci2 = ci0 + (k + _NIN - 1) // _B
            b2 = (k + _NIN - 1) % _B

            @pl.when(t + _NIN - 1 < _NT)
            def _():
                pltpu.async_copy(x_src(ci2, b2), x_bufs[s2], in_sems[s2])

            if k % _B == _B - 1:
                @pl.when(ci + 2 < _NCHUNK)
                def _():
                    pltpu.async_copy(pos_src(ci + 2), pos_bufs[pb],
                                     pos_sems[pb])

    last = _NCHUNK - 1
    for j in range(_NOUT):
        pltpu.make_async_copy(o_bufs[j], out_dst(last, j),
                              out_sems[j]).wait()


_sc_call = functools.partial(
    pl.kernel,
    out_type=jax.ShapeDtypeStruct((_B, _S, _D), jnp.float32),
    mesh=plsc.VectorSubcoreMesh(
        core_axis_name="c", subcore_axis_name="s",
        num_cores=_NC, num_subcores=_NS,
    ),
    scratch_types=(
        [pltpu.VMEM((_CH, _D), jnp.float32)] * (2 + _NIN + _NOUT)
        + [pltpu.SemaphoreType.DMA] * (2 + _NIN + _NOUT)
    ),
    compiler_params=pltpu.CompilerParams(use_tc_tiling_on_sc=True),
)(_sc_body)


def kernel(x, pos_table):
    B, S, D = x.shape
    return _sc_call(x, pos_table[:S])
